# Initial kernel scaffold; baseline (speedup 1.0000x reference)
import jax
import jax.numpy as jnp
from jax.experimental import pallas as pl


def kernel(x, edge_index, W1, b1, W2, b2):
    raise NotImplementedError("write your pallas kernel here")


# speedup vs baseline: 6.0140x; 6.0140x over previous
import functools

import jax
import jax.numpy as jnp
from jax import lax
from jax.experimental import pallas as pl
from jax.experimental.pallas import tpu as pltpu
from jax.experimental.pallas import tpu_sc as plsc

NC = 2
NS = 16
NW = NC * NS
CK = 128
RB = 1000


def _mesh():
    return plsc.VectorSubcoreMesh(
        core_axis_name="c", subcore_axis_name="s", num_cores=NC, num_subcores=NS
    )


def _edge_aggregate(y, src3, dst3, n_pad):
    d = y.shape[1]
    nchunk = src3.shape[1]
    zps = n_pad // NS
    zeros = jnp.zeros((n_pad, d), jnp.float32)
    ib = 8

    @functools.partial(
        pl.kernel,
        mesh=_mesh(),
        out_type=jax.ShapeDtypeStruct((NC, n_pad, d), jnp.float32),
        scratch_types=[
            pltpu.VMEM((ib, CK), jnp.int32),
            pltpu.VMEM((ib, CK), jnp.int32),
            pltpu.VMEM((CK, d), jnp.float32),
            pltpu.VMEM((CK, d), jnp.float32),
            pltpu.VMEM_SHARED((n_pad, d), jnp.float32),
            pltpu.SemaphoreType.DMA,
            pltpu.SemaphoreType.DMA,
        ],
    )
    def scat_k(y_hbm, src_hbm, dst_hbm, zeros_hbm, out_hbm,
               si_v, di_v, r_a, r_b, acc, sem_a, sem_b):
        c = lax.axis_index("c")
        s = lax.axis_index("s")
        wid = s * NC + c
        pltpu.sync_copy(zeros_hbm.at[pl.ds(s * zps, zps)], acc.at[pl.ds(s * zps, zps)])
        plsc.subcore_barrier()

        def outer(g, carry):
            pltpu.sync_copy(src_hbm.at[wid, pl.ds(g * ib, ib)], si_v)
            pltpu.sync_copy(dst_hbm.at[wid, pl.ds(g * ib, ib)], di_v)

            def body(i, c2):
                k = 2 * i
                g_a = pltpu.async_copy(y_hbm.at[si_v.at[k]], r_a, sem_a)
                g_b = pltpu.async_copy(y_hbm.at[si_v.at[k + 1]], r_b, sem_b)
                g_a.wait()
                pltpu.sync_copy(r_a, acc.at[di_v.at[k]], add=True)
                g_b.wait()
                pltpu.sync_copy(r_b, acc.at[di_v.at[k + 1]], add=True)
                return c2

            lax.fori_loop(0, ib // 2, body, 0)
            return carry

        lax.fori_loop(0, nchunk // ib, outer, 0)
        plsc.subcore_barrier()
        pltpu.sync_copy(acc.at[pl.ds(s * zps, zps)], out_hbm.at[c, pl.ds(s * zps, zps)])

    return scat_k(y, src3, dst3, zeros)


def _disb_body(dp_ref, o_ref):
    o_ref[...] = lax.rsqrt(dp_ref[0] + dp_ref[1] + 1.0)


def _disb_kernel(degp, n, d):
    return pl.pallas_call(
        _disb_body,
        grid=(n // RB,),
        in_specs=[pl.BlockSpec((NC, RB, d), lambda i: (0, i, 0))],
        out_specs=pl.BlockSpec((RB, d), lambda i: (i, 0)),
        out_shape=jax.ShapeDtypeStruct((n, d), jnp.float32),
    )(degp)


def _matmul(a_ref, w_ref):
    return lax.dot_general(
        a_ref, w_ref, (((1,), (0,)), ((), ())),
        precision=lax.Precision.HIGHEST, preferred_element_type=jnp.float32,
    )


def _y1_body(x_ref, w_ref, d_ref, o_ref):
    o_ref[...] = _matmul(x_ref[...], w_ref[...]) * d_ref[...]


def _y1_kernel(x, W1, disb):
    n, d = x.shape
    return pl.pallas_call(
        _y1_body,
        grid=(n // RB,),
        in_specs=[
            pl.BlockSpec((RB, d), lambda i: (i, 0)),
            pl.BlockSpec((d, d), lambda i: (0, 0)),
            pl.BlockSpec((RB, d), lambda i: (i, 0)),
        ],
        out_specs=pl.BlockSpec((RB, d), lambda i: (i, 0)),
        out_shape=jax.ShapeDtypeStruct((n, d), jnp.float32),
    )(x, W1, disb)


def _y2_body(a_ref, y1_ref, d_ref, b_ref, w_ref, o_ref):
    h = d_ref[...] * (a_ref[0] + a_ref[1] + y1_ref[...]) + b_ref[...]
    h = jnp.maximum(h, 0.0)
    o_ref[...] = _matmul(h, w_ref[...]) * d_ref[...]


def _y2_kernel(a1, y1, disb, b1, W2):
    n, d = y1.shape
    return pl.pallas_call(
        _y2_body,
        grid=(n // RB,),
        in_specs=[
            pl.BlockSpec((NC, RB, d), lambda i: (0, i, 0)),
            pl.BlockSpec((RB, d), lambda i: (i, 0)),
            pl.BlockSpec((RB, d), lambda i: (i, 0)),
            pl.BlockSpec((1, d), lambda i: (0, 0)),
            pl.BlockSpec((d, d), lambda i: (0, 0)),
        ],
        out_specs=pl.BlockSpec((RB, d), lambda i: (i, 0)),
        out_shape=jax.ShapeDtypeStruct((n, d), jnp.float32),
    )(a1, y1, disb, b1, W2)


def _out_body(a_ref, y2_ref, d_ref, b_ref, o_ref):
    o_ref[...] = d_ref[...] * (a_ref[0] + a_ref[1] + y2_ref[...]) + b_ref[...]


def _out_kernel(a2, y2, disb, b2):
    n, d = y2.shape
    return pl.pallas_call(
        _out_body,
        grid=(n // RB,),
        in_specs=[
            pl.BlockSpec((NC, RB, d), lambda i: (0, i, 0)),
            pl.BlockSpec((RB, d), lambda i: (i, 0)),
            pl.BlockSpec((RB, d), lambda i: (i, 0)),
            pl.BlockSpec((1, d), lambda i: (0, 0)),
        ],
        out_specs=pl.BlockSpec((RB, d), lambda i: (i, 0)),
        out_shape=jax.ShapeDtypeStruct((n, d), jnp.float32),
    )(a2, y2, disb, b2)


def kernel(x, edge_index, W1, b1, W2, b2):
    n, d = x.shape
    e = edge_index.shape[1]

    src = edge_index[0].astype(jnp.int32)
    dst = edge_index[1].astype(jnp.int32)

    epair = NW * CK * 2
    e_pad = ((e + epair - 1) // epair) * epair
    pad = e_pad - e
    src_p = jnp.concatenate([src, jnp.zeros((pad,), jnp.int32)])
    dst_p = jnp.concatenate([dst, jnp.full((pad,), n, jnp.int32)])
    nchunk = e_pad // NW // CK
    src3 = src_p.reshape(NW, nchunk, CK)
    dst3 = dst_p.reshape(NW, nchunk, CK)

    n_pad = ((n + 1 + 127) // 128) * 128

    ones = jnp.ones((n_pad, d), jnp.float32)
    degp = _edge_aggregate(ones, dst3, dst3, n_pad)
    disb = _disb_kernel(degp, n, d)
    y1 = _y1_kernel(x, W1, disb)
    a1 = _edge_aggregate(y1, src3, dst3, n_pad)
    y2 = _y2_kernel(a1, y1, disb, b1.reshape(1, d), W2)
    a2 = _edge_aggregate(y2, src3, dst3, n_pad)
    return _out_kernel(a2, y2, disb, b2.reshape(1, d))


# speedup vs baseline: 6.7334x; 1.1196x over previous
import functools

import jax
import jax.numpy as jnp
from jax import lax
from jax.experimental import pallas as pl
from jax.experimental.pallas import tpu as pltpu
from jax.experimental.pallas import tpu_sc as plsc

NC = 2
NS = 16
NW = NC * NS
CK = 128
RB = 1000


def _mesh():
    return plsc.VectorSubcoreMesh(
        core_axis_name="c", subcore_axis_name="s", num_cores=NC, num_subcores=NS
    )


def _edge_aggregate(y, src3, dst3, n_pad):
    d = y.shape[1]
    nchunk = src3.shape[1]
    zps = n_pad // NS
    zeros = jnp.zeros((n_pad, d), jnp.float32)
    ib = nchunk // 2

    @functools.partial(
        pl.kernel,
        mesh=_mesh(),
        out_type=jax.ShapeDtypeStruct((NC, n_pad, d), jnp.float32),
        scratch_types=[
            pltpu.VMEM((ib, CK), jnp.int32),
            pltpu.VMEM((ib, CK), jnp.int32),
            pltpu.VMEM((CK, d), jnp.float32),
            pltpu.VMEM((CK, d), jnp.float32),
            pltpu.VMEM_SHARED((n_pad, d), jnp.float32),
            pltpu.SemaphoreType.DMA,
            pltpu.SemaphoreType.DMA,
        ],
    )
    def scat_k(y_hbm, src_hbm, dst_hbm, zeros_hbm, out_hbm,
               si_v, di_v, r_a, r_b, acc, sem_a, sem_b):
        c = lax.axis_index("c")
        s = lax.axis_index("s")
        wid = s * NC + c
        pltpu.sync_copy(zeros_hbm.at[pl.ds(s * zps, zps)], acc.at[pl.ds(s * zps, zps)])
        plsc.subcore_barrier()

        def fire(k, rbuf, sem):
            pltpu.async_copy(y_hbm.at[si_v.at[k]], rbuf, sem)

        def drain(rbuf, sem):
            pltpu.make_async_copy(y_hbm.at[si_v.at[0]], rbuf, sem).wait()

        def scat(k, rbuf):
            pltpu.sync_copy(rbuf, acc.at[di_v.at[k]], add=True)

        for g in range(nchunk // ib):
            pltpu.sync_copy(src_hbm.at[wid, pl.ds(g * ib, ib)], si_v)
            pltpu.sync_copy(dst_hbm.at[wid, pl.ds(g * ib, ib)], di_v)
            fire(0, r_a, sem_a)
            fire(1, r_b, sem_b)

            def body(i, c2):
                k = 2 * i
                drain(r_a, sem_a)
                scat(k, r_a)
                fire(k + 2, r_a, sem_a)
                drain(r_b, sem_b)
                scat(k + 1, r_b)
                fire(k + 3, r_b, sem_b)
                return c2

            lax.fori_loop(0, ib // 2 - 1, body, 0)
            drain(r_a, sem_a)
            scat(ib - 2, r_a)
            drain(r_b, sem_b)
            scat(ib - 1, r_b)

        plsc.subcore_barrier()
        pltpu.sync_copy(acc.at[pl.ds(s * zps, zps)], out_hbm.at[c, pl.ds(s * zps, zps)])

    return scat_k(y, src3, dst3, zeros)


def _disb_body(dp_ref, o_ref):
    o_ref[...] = lax.rsqrt(dp_ref[0] + dp_ref[1] + 1.0)


def _disb_kernel(degp, n, d):
    return pl.pallas_call(
        _disb_body,
        grid=(n // RB,),
        in_specs=[pl.BlockSpec((NC, RB, d), lambda i: (0, i, 0))],
        out_specs=pl.BlockSpec((RB, d), lambda i: (i, 0)),
        out_shape=jax.ShapeDtypeStruct((n, d), jnp.float32),
    )(degp)


def _matmul(a_ref, w_ref):
    return lax.dot_general(
        a_ref, w_ref, (((1,), (0,)), ((), ())),
        precision=lax.Precision.HIGHEST, preferred_element_type=jnp.float32,
    )


def _y1_body(x_ref, w_ref, d_ref, o_ref):
    o_ref[...] = _matmul(x_ref[...], w_ref[...]) * d_ref[...]


def _y1_kernel(x, W1, disb):
    n, d = x.shape
    return pl.pallas_call(
        _y1_body,
        grid=(n // RB,),
        in_specs=[
            pl.BlockSpec((RB, d), lambda i: (i, 0)),
            pl.BlockSpec((d, d), lambda i: (0, 0)),
            pl.BlockSpec((RB, d), lambda i: (i, 0)),
        ],
        out_specs=pl.BlockSpec((RB, d), lambda i: (i, 0)),
        out_shape=jax.ShapeDtypeStruct((n, d), jnp.float32),
    )(x, W1, disb)


def _y2_body(a_ref, y1_ref, d_ref, b_ref, w_ref, o_ref):
    h = d_ref[...] * (a_ref[0] + a_ref[1] + y1_ref[...]) + b_ref[...]
    h = jnp.maximum(h, 0.0)
    o_ref[...] = _matmul(h, w_ref[...]) * d_ref[...]


def _y2_kernel(a1, y1, disb, b1, W2):
    n, d = y1.shape
    return pl.pallas_call(
        _y2_body,
        grid=(n // RB,),
        in_specs=[
            pl.BlockSpec((NC, RB, d), lambda i: (0, i, 0)),
            pl.BlockSpec((RB, d), lambda i: (i, 0)),
            pl.BlockSpec((RB, d), lambda i: (i, 0)),
            pl.BlockSpec((1, d), lambda i: (0, 0)),
            pl.BlockSpec((d, d), lambda i: (0, 0)),
        ],
        out_specs=pl.BlockSpec((RB, d), lambda i: (i, 0)),
        out_shape=jax.ShapeDtypeStruct((n, d), jnp.float32),
    )(a1, y1, disb, b1, W2)


def _out_body(a_ref, y2_ref, d_ref, b_ref, o_ref):
    o_ref[...] = d_ref[...] * (a_ref[0] + a_ref[1] + y2_ref[...]) + b_ref[...]


def _out_kernel(a2, y2, disb, b2):
    n, d = y2.shape
    return pl.pallas_call(
        _out_body,
        grid=(n // RB,),
        in_specs=[
            pl.BlockSpec((NC, RB, d), lambda i: (0, i, 0)),
            pl.BlockSpec((RB, d), lambda i: (i, 0)),
            pl.BlockSpec((RB, d), lambda i: (i, 0)),
            pl.BlockSpec((1, d), lambda i: (0, 0)),
        ],
        out_specs=pl.BlockSpec((RB, d), lambda i: (i, 0)),
        out_shape=jax.ShapeDtypeStruct((n, d), jnp.float32),
    )(a2, y2, disb, b2)


def kernel(x, edge_index, W1, b1, W2, b2):
    n, d = x.shape
    e = edge_index.shape[1]

    src = edge_index[0].astype(jnp.int32)
    dst = edge_index[1].astype(jnp.int32)

    epair = NW * CK * 2
    e_pad = ((e + epair - 1) // epair) * epair
    pad = e_pad - e
    src_p = jnp.concatenate([src, jnp.zeros((pad,), jnp.int32)])
    dst_p = jnp.concatenate([dst, jnp.full((pad,), n, jnp.int32)])
    nchunk = e_pad // NW // CK
    src3 = src_p.reshape(NW, nchunk, CK)
    dst3 = dst_p.reshape(NW, nchunk, CK)

    n_pad = ((n + 1 + 127) // 128) * 128

    ones = jnp.ones((n_pad, d), jnp.float32)
    degp = _edge_aggregate(ones, dst3, dst3, n_pad)
    disb = _disb_kernel(degp, n, d)
    y1 = _y1_kernel(x, W1, disb)
    a1 = _edge_aggregate(y1, src3, dst3, n_pad)
    y2 = _y2_kernel(a1, y1, disb, b1.reshape(1, d), W2)
    a2 = _edge_aggregate(y2, src3, dst3, n_pad)
    return _out_kernel(a2, y2, disb, b2.reshape(1, d))


# speedup vs baseline: 8.8631x; 1.3163x over previous
import functools

import jax
import jax.numpy as jnp
from jax import lax
from jax.experimental import pallas as pl
from jax.experimental.pallas import tpu as pltpu
from jax.experimental.pallas import tpu_sc as plsc

NC = 2
NS = 16
NW = NC * NS
CK = 128
RB = 1000


def _mesh():
    return plsc.VectorSubcoreMesh(
        core_axis_name="c", subcore_axis_name="s", num_cores=NC, num_subcores=NS
    )


def _edge_aggregate(y, src3, dst3, n_pad):
    d = y.shape[1]
    nchunk = src3.shape[1]
    zps = n_pad // NS
    zeros = jnp.zeros((n_pad, d), jnp.float32)
    ib = nchunk // 2

    @functools.partial(
        pl.kernel,
        mesh=_mesh(),
        out_type=jax.ShapeDtypeStruct((NC, n_pad, d), jnp.float32),
        scratch_types=[
            pltpu.VMEM((ib, CK), jnp.int32),
            pltpu.VMEM((ib, CK), jnp.int32),
            pltpu.VMEM((CK, d), jnp.float32),
            pltpu.VMEM((CK, d), jnp.float32),
            pltpu.VMEM_SHARED((n_pad, d), jnp.float32),
            pltpu.SemaphoreType.DMA,
            pltpu.SemaphoreType.DMA,
        ],
    )
    def scat_k(y_hbm, src_hbm, dst_hbm, zeros_hbm, out_hbm,
               si_v, di_v, r_a, r_b, acc, sem_a, sem_b):
        c = lax.axis_index("c")
        s = lax.axis_index("s")
        wid = s * NC + c
        pltpu.sync_copy(zeros_hbm.at[pl.ds(s * zps, zps)], acc.at[pl.ds(s * zps, zps)])
        plsc.subcore_barrier()

        def fire(k, rbuf, sem):
            pltpu.async_copy(y_hbm.at[si_v.at[k]], rbuf, sem)

        def drain(rbuf, sem):
            pltpu.make_async_copy(y_hbm.at[si_v.at[0]], rbuf, sem).wait()

        def scat(k, rbuf):
            pltpu.sync_copy(rbuf, acc.at[di_v.at[k]], add=True)

        for g in range(nchunk // ib):
            pltpu.sync_copy(src_hbm.at[wid, pl.ds(g * ib, ib)], si_v)
            pltpu.sync_copy(dst_hbm.at[wid, pl.ds(g * ib, ib)], di_v)
            fire(0, r_a, sem_a)
            fire(1, r_b, sem_b)

            def body(i, c2):
                k = 2 * i
                drain(r_a, sem_a)
                scat(k, r_a)
                fire(k + 2, r_a, sem_a)
                drain(r_b, sem_b)
                scat(k + 1, r_b)
                fire(k + 3, r_b, sem_b)
                return c2

            lax.fori_loop(0, ib // 2 - 1, body, 0)
            drain(r_a, sem_a)
            scat(ib - 2, r_a)
            drain(r_b, sem_b)
            scat(ib - 1, r_b)

        plsc.subcore_barrier()
        pltpu.sync_copy(acc.at[pl.ds(s * zps, zps)], out_hbm.at[c, pl.ds(s * zps, zps)])

    return scat_k(y, src3, dst3, zeros)


def _deg_aggregate(dst3, n_pad, d):
    nchunk = dst3.shape[1]
    zps = n_pad // NS
    zeros = jnp.zeros((n_pad, d), jnp.float32)
    ones = jnp.ones((CK, d), jnp.float32)
    ib = nchunk // 2

    @functools.partial(
        pl.kernel,
        mesh=_mesh(),
        out_type=jax.ShapeDtypeStruct((NC, n_pad, d), jnp.float32),
        scratch_types=[
            pltpu.VMEM((ib, CK), jnp.int32),
            pltpu.VMEM((CK, d), jnp.float32),
            pltpu.VMEM((CK, d), jnp.float32),
            pltpu.VMEM_SHARED((n_pad, d), jnp.float32),
            pltpu.SemaphoreType.DMA,
            pltpu.SemaphoreType.DMA,
        ],
    )
    def deg_k(dst_hbm, zeros_hbm, ones_hbm, out_hbm,
              di_v, r_a, r_b, acc, sem_a, sem_b):
        c = lax.axis_index("c")
        s = lax.axis_index("s")
        wid = s * NC + c
        pltpu.sync_copy(zeros_hbm.at[pl.ds(s * zps, zps)], acc.at[pl.ds(s * zps, zps)])
        pltpu.sync_copy(ones_hbm, r_a)
        pltpu.sync_copy(ones_hbm, r_b)
        plsc.subcore_barrier()

        for g in range(nchunk // ib):
            pltpu.sync_copy(dst_hbm.at[wid, pl.ds(g * ib, ib)], di_v)

            def body(i, c2):
                k = 2 * i
                s_a = pltpu.async_copy(r_a, acc.at[di_v.at[k]], sem_a, add=True)
                s_b = pltpu.async_copy(r_b, acc.at[di_v.at[k + 1]], sem_b, add=True)
                s_a.wait()
                s_b.wait()
                return c2

            lax.fori_loop(0, ib // 2, body, 0)

        plsc.subcore_barrier()
        pltpu.sync_copy(acc.at[pl.ds(s * zps, zps)], out_hbm.at[c, pl.ds(s * zps, zps)])

    return deg_k(dst3, zeros, ones)


def _disb_body(dp_ref, o_ref):
    o_ref[...] = lax.rsqrt(dp_ref[0] + dp_ref[1] + 1.0)


def _disb_kernel(degp, n, d):
    return pl.pallas_call(
        _disb_body,
        grid=(n // RB,),
        in_specs=[pl.BlockSpec((NC, RB, d), lambda i: (0, i, 0))],
        out_specs=pl.BlockSpec((RB, d), lambda i: (i, 0)),
        out_shape=jax.ShapeDtypeStruct((n, d), jnp.float32),
    )(degp)


def _matmul(a_ref, w_ref):
    return lax.dot_general(
        a_ref, w_ref, (((1,), (0,)), ((), ())),
        precision=lax.Precision.HIGHEST, preferred_element_type=jnp.float32,
    )


def _y1_body(x_ref, w_ref, d_ref, o_ref):
    o_ref[...] = _matmul(x_ref[...], w_ref[...]) * d_ref[...]


def _y1_kernel(x, W1, disb):
    n, d = x.shape
    return pl.pallas_call(
        _y1_body,
        grid=(n // RB,),
        in_specs=[
            pl.BlockSpec((RB, d), lambda i: (i, 0)),
            pl.BlockSpec((d, d), lambda i: (0, 0)),
            pl.BlockSpec((RB, d), lambda i: (i, 0)),
        ],
        out_specs=pl.BlockSpec((RB, d), lambda i: (i, 0)),
        out_shape=jax.ShapeDtypeStruct((n, d), jnp.float32),
    )(x, W1, disb)


def _y2_body(a_ref, y1_ref, d_ref, b_ref, w_ref, o_ref):
    h = d_ref[...] * (a_ref[0] + a_ref[1] + y1_ref[...]) + b_ref[...]
    h = jnp.maximum(h, 0.0)
    o_ref[...] = _matmul(h, w_ref[...]) * d_ref[...]


def _y2_kernel(a1, y1, disb, b1, W2):
    n, d = y1.shape
    return pl.pallas_call(
        _y2_body,
        grid=(n // RB,),
        in_specs=[
            pl.BlockSpec((NC, RB, d), lambda i: (0, i, 0)),
            pl.BlockSpec((RB, d), lambda i: (i, 0)),
            pl.BlockSpec((RB, d), lambda i: (i, 0)),
            pl.BlockSpec((1, d), lambda i: (0, 0)),
            pl.BlockSpec((d, d), lambda i: (0, 0)),
        ],
        out_specs=pl.BlockSpec((RB, d), lambda i: (i, 0)),
        out_shape=jax.ShapeDtypeStruct((n, d), jnp.float32),
    )(a1, y1, disb, b1, W2)


def _out_body(a_ref, y2_ref, d_ref, b_ref, o_ref):
    o_ref[...] = d_ref[...] * (a_ref[0] + a_ref[1] + y2_ref[...]) + b_ref[...]


def _out_kernel(a2, y2, disb, b2):
    n, d = y2.shape
    return pl.pallas_call(
        _out_body,
        grid=(n // RB,),
        in_specs=[
            pl.BlockSpec((NC, RB, d), lambda i: (0, i, 0)),
            pl.BlockSpec((RB, d), lambda i: (i, 0)),
            pl.BlockSpec((RB, d), lambda i: (i, 0)),
            pl.BlockSpec((1, d), lambda i: (0, 0)),
        ],
        out_specs=pl.BlockSpec((RB, d), lambda i: (i, 0)),
        out_shape=jax.ShapeDtypeStruct((n, d), jnp.float32),
    )(a2, y2, disb, b2)


def kernel(x, edge_index, W1, b1, W2, b2):
    n, d = x.shape
    e = edge_index.shape[1]

    src = edge_index[0].astype(jnp.int32)
    dst = edge_index[1].astype(jnp.int32)

    epair = NW * CK * 2
    e_pad = ((e + epair - 1) // epair) * epair
    pad = e_pad - e
    src_p = jnp.concatenate([src, jnp.zeros((pad,), jnp.int32)])
    dst_p = jnp.concatenate([dst, jnp.full((pad,), n, jnp.int32)])
    nchunk = e_pad // NW // CK
    src3 = src_p.reshape(NW, nchunk, CK)
    dst3 = dst_p.reshape(NW, nchunk, CK)

    n_pad = ((n + 1 + 127) // 128) * 128

    degp = _deg_aggregate(dst3, n_pad, d)
    disb = _disb_kernel(degp, n, d)
    y1 = _y1_kernel(x, W1, disb)
    a1 = _edge_aggregate(y1, src3, dst3, n_pad)
    y2 = _y2_kernel(a1, y1, disb, b1.reshape(1, d), W2)
    a2 = _edge_aggregate(y2, src3, dst3, n_pad)
    return _out_kernel(a2, y2, disb, b2.reshape(1, d))
